# baseline (speedup 1.0000x reference)
import jax
import jax.numpy as jnp
from jax import lax
from jax.experimental import pallas as pl
from jax.experimental.pallas import tpu as pltpu
from jax.experimental.pallas import tpu_sc as plsc

_NC = 2
_NS = 16
_NW = _NC * _NS


def _sc_gather_body(x2, gall, sel2, rem2, idx_v, buf0, buf1,
                    gs0, gs1, os0, os1, rows_pw, row_el):
    wid = lax.axis_index("s") * _NC + lax.axis_index("c")
    pltpu.sync_copy(gall.at[pl.ds(wid * rows_pw, rows_pw)], idx_v)
    half = wid // _NS
    obase = (wid % _NS) * rows_pw
    bufs = (buf0, buf1)
    gsems = (gs0, gs1)
    osems = (os0, os1)

    def run(out2):
        pltpu.async_copy(x2.at[idx_v.at[0]], bufs[0], gsems[0])
        pltpu.async_copy(x2.at[idx_v.at[1]], bufs[1], gsems[1])
        for k in range(rows_pw):
            b = k % 2
            pltpu.make_async_copy(
                x2.at[idx_v.at[k]], bufs[b], gsems[b]).wait()
            dst = out2.at[pl.ds(obase + k, 1)]
            pltpu.async_copy(bufs[b], dst, osems[b])
            if k + 2 < rows_pw:
                pltpu.make_async_copy(bufs[b], dst, osems[b]).wait()
                pltpu.async_copy(
                    x2.at[idx_v.at[k + 2]], bufs[b], gsems[b])
        for k in (rows_pw - 2, rows_pw - 1):
            b = k % 2
            pltpu.make_async_copy(
                bufs[b], out2.at[pl.ds(obase + k, 1)], osems[b]).wait()

    @pl.when(half == 0)
    def _():
        run(sel2)

    @pl.when(half == 1)
    def _():
        run(rem2)


def kernel(x, W1, b1, W2, b2):
    N, C, H, W = x.shape
    rc = C // 2
    row_el = H * W
    rows_pw = (N * C) // _NW
    z = jnp.mean(x, axis=(2, 3))
    s = jax.nn.relu(z @ W1.T + b1)
    s = jax.nn.sigmoid(s @ W2.T + b2)
    indices = jnp.argsort(-s, axis=1).astype(jnp.int32)

    rows = jnp.arange(N, dtype=jnp.int32)[:, None] * C + indices
    gall = jnp.concatenate(
        [rows[:, :rc].reshape(-1), rows[:, rc:].reshape(-1)])[:, None]

    x2 = x.reshape(N * C, row_el)
    import functools
    body = functools.partial(_sc_gather_body, rows_pw=rows_pw, row_el=row_el)
    sel2, rem2 = pl.kernel(
        body,
        out_type=[
            jax.ShapeDtypeStruct((N * rc, row_el), x.dtype),
            jax.ShapeDtypeStruct((N * (C - rc), row_el), x.dtype),
        ],
        mesh=plsc.VectorSubcoreMesh(core_axis_name="c", subcore_axis_name="s"),
        scratch_types=[
            pltpu.VMEM((rows_pw, 1), jnp.int32),
            pltpu.VMEM((1, row_el), jnp.float32),
            pltpu.VMEM((1, row_el), jnp.float32),
            pltpu.SemaphoreType.DMA,
            pltpu.SemaphoreType.DMA,
            pltpu.SemaphoreType.DMA,
            pltpu.SemaphoreType.DMA,
        ],
    )(x2, gall)
    sel = sel2.reshape(N, rc, H, W)
    rem = rem2.reshape(N, C - rc, H, W)
    return sel, rem
